# baseline (speedup 1.0000x reference)
import jax
import jax.numpy as jnp
from jax.experimental import pallas as pl
from jax.experimental.pallas import tpu as pltpu
from jax.experimental.pallas import tpu_sc as plsc

_B = 16384
_D = 64
_NS = 15
_NPAD = 16
_BLK = 2048


def _sc_gather(emb_pairs, idxs):
    mesh = plsc.VectorSubcoreMesh(core_axis_name="c", subcore_axis_name="s")

    @pl.kernel(
        out_type=jax.ShapeDtypeStruct((_NPAD, 2 * _D), emb_pairs.dtype),
        mesh=mesh,
    )
    def gather_kernel(emb_hbm, idx_hbm, out_hbm):
        def body(i_vmem, o_vmem):
            pltpu.sync_copy(emb_hbm.at[i_vmem.at[0]], o_vmem)

        pltpu.emit_pipeline(
            body,
            grid=(1,),
            in_specs=[pl.BlockSpec((1, _NPAD), lambda i: (0, 0))],
            out_specs=[pl.BlockSpec((_NPAD, 2 * _D), lambda i: (0, 0))],
            core_axis_name="s",
            dimension_semantics=(pltpu.PARALLEL,),
        )(idx_hbm, out_hbm)

    return gather_kernel(emb_pairs, idxs)


def _logsig(x):
    return jnp.minimum(x, 0.0) - jnp.log1p(jnp.exp(-jnp.abs(x)))


def _loss_body(center_ref, context_ref, samples_ref, half_ref, out_ref):
    i = pl.program_id(0)
    c = center_ref[...]
    t = context_ref[...]
    w = samples_ref[...]
    h = half_ref[...]
    s = w[:, :_D] * (1.0 - h) + w[:, _D:] * h

    pos = jnp.sum(c * t, axis=1)
    pos_ls = _logsig(pos)

    dots = jax.lax.dot_general(
        c, s, (((1,), (1,)), ((), ())), preferred_element_type=jnp.float32
    )
    neg_ls = _logsig(-dots)
    col = jax.lax.broadcasted_iota(jnp.int32, neg_ls.shape, 1)
    neg_ls = jnp.where(col < _NS, neg_ls, 0.0)

    part = jnp.sum(pos_ls) + jnp.sum(neg_ls)

    @pl.when(i == 0)
    def _():
        out_ref[0, 0] = 0.0

    out_ref[0, 0] += -part

    @pl.when(i == (_B // _BLK) - 1)
    def _():
        out_ref[0, 0] = out_ref[0, 0] * (1.0 / _B)


def kernel(center, context, neg_idxs, emb_table):
    idx = neg_idxs.astype(jnp.int32)
    idx = jnp.concatenate([idx, idx[:1]])
    pair = (idx // 2).reshape(1, _NPAD)
    half = (idx % 2).astype(jnp.float32).reshape(_NPAD, 1)
    emb_pairs = emb_table.reshape(emb_table.shape[0] // 2, 2 * _D)
    samples = _sc_gather(emb_pairs, pair)

    nb = _B // _BLK
    out = pl.pallas_call(
        _loss_body,
        grid=(nb,),
        in_specs=[
            pl.BlockSpec((_BLK, _D), lambda i: (i, 0)),
            pl.BlockSpec((_BLK, _D), lambda i: (i, 0)),
            pl.BlockSpec((_NPAD, 2 * _D), lambda i: (0, 0)),
            pl.BlockSpec((_NPAD, 1), lambda i: (0, 0)),
        ],
        out_specs=pl.BlockSpec(
            (1, 1), lambda i: (0, 0), memory_space=pltpu.SMEM
        ),
        out_shape=jax.ShapeDtypeStruct((1, 1), jnp.float32),
    )(center, context, samples, half)
    return out[0, 0]


# speedup vs baseline: 1.3480x; 1.3480x over previous
import jax
import jax.numpy as jnp
from jax.experimental import pallas as pl
from jax.experimental.pallas import tpu as pltpu
from jax.experimental.pallas import tpu_sc as plsc

_B = 16384
_D = 64
_NS = 15
_NPAD = 16
_BLK = 2048
_NCORES = 2


def _sc_gather(emb_table, idxs):
    mesh = plsc.ScalarSubcoreMesh(axis_name="core", num_cores=_NCORES)
    per_core = _NPAD // _NCORES

    @pl.kernel(
        out_type=jax.ShapeDtypeStruct((_NPAD, _D), emb_table.dtype),
        mesh=mesh,
        scratch_types=[
            pltpu.SMEM((1, _NPAD), jnp.int32),
            pltpu.SemaphoreType.DMA,
        ],
    )
    def gather_kernel(emb_hbm, idx_hbm, out_hbm, idx_smem, sem):
        core = jax.lax.axis_index("core")
        pltpu.async_copy(idx_hbm, idx_smem, sem).wait()
        copies = []
        for n in range(per_core):
            row = core * per_core + n
            i = idx_smem[0, row]
            cp = pltpu.make_async_copy(emb_hbm.at[i], out_hbm.at[row], sem)
            cp.start()
            copies.append(cp)
        for cp in copies:
            cp.wait()

    return gather_kernel(emb_table, idxs)


def _logsig(x):
    return jnp.minimum(x, 0.0) - jnp.log1p(jnp.exp(-jnp.abs(x)))


def _loss_body(center_ref, context_ref, samples_ref, out_ref):
    i = pl.program_id(0)
    c = center_ref[...]
    t = context_ref[...]
    s = samples_ref[...]

    pos = jnp.sum(c * t, axis=1)
    pos_ls = _logsig(pos)

    dots = jax.lax.dot_general(
        c, s, (((1,), (1,)), ((), ())), preferred_element_type=jnp.float32
    )
    neg_ls = _logsig(-dots)
    col = jax.lax.broadcasted_iota(jnp.int32, neg_ls.shape, 1)
    neg_ls = jnp.where(col < _NS, neg_ls, 0.0)

    part = jnp.sum(pos_ls) + jnp.sum(neg_ls)

    @pl.when(i == 0)
    def _():
        out_ref[0, 0] = 0.0

    out_ref[0, 0] += -part

    @pl.when(i == (_B // _BLK) - 1)
    def _():
        out_ref[0, 0] = out_ref[0, 0] * (1.0 / _B)


def kernel(center, context, neg_idxs, emb_table):
    idx = neg_idxs.astype(jnp.int32)
    idx = jnp.concatenate([idx, idx[:1]]).reshape(1, _NPAD)
    samples = _sc_gather(emb_table, idx)

    nb = _B // _BLK
    out = pl.pallas_call(
        _loss_body,
        grid=(nb,),
        in_specs=[
            pl.BlockSpec((_BLK, _D), lambda i: (i, 0)),
            pl.BlockSpec((_BLK, _D), lambda i: (i, 0)),
            pl.BlockSpec((_NPAD, _D), lambda i: (0, 0)),
        ],
        out_specs=pl.BlockSpec(
            (1, 1), lambda i: (0, 0), memory_space=pltpu.SMEM
        ),
        out_shape=jax.ShapeDtypeStruct((1, 1), jnp.float32),
    )(center, context, samples)
    return out[0, 0]


# speedup vs baseline: 1.4347x; 1.0644x over previous
import jax
import jax.numpy as jnp
from jax.experimental import pallas as pl
from jax.experimental.pallas import tpu as pltpu
from jax.experimental.pallas import tpu_sc as plsc

_B = 16384
_D = 64
_NS = 15
_NPAD = 16
_BLK = 2048
_NCORES = 2


def _sc_gather(emb_table, idxs):
    mesh = plsc.ScalarSubcoreMesh(axis_name="core", num_cores=_NCORES)
    per_core = _NPAD // _NCORES

    @pl.kernel(
        out_type=jax.ShapeDtypeStruct((_NPAD, _D), emb_table.dtype),
        mesh=mesh,
        scratch_types=[
            pltpu.SMEM((1, _NPAD), jnp.int32),
            pltpu.SemaphoreType.DMA,
        ],
    )
    def gather_kernel(emb_hbm, idx_hbm, out_hbm, idx_smem, sem):
        core = jax.lax.axis_index("core")
        pltpu.async_copy(idx_hbm, idx_smem, sem).wait()
        copies = []
        for n in range(per_core):
            row = core * per_core + n
            i = idx_smem[0, row]
            cp = pltpu.make_async_copy(emb_hbm.at[i], out_hbm.at[row], sem)
            cp.start()
            copies.append(cp)
        for cp in copies:
            cp.wait()

    return gather_kernel(emb_table, idxs)


def _logsig(x):
    return jnp.minimum(x, 0.0) - jnp.log1p(jnp.exp(-jnp.abs(x)))


def _loss_body(center_ref, context_ref, samples_ref, out_ref):
    i = pl.program_id(0)
    c = center_ref[...]
    t = context_ref[...]
    s = samples_ref[...]

    pos = jnp.sum(c * t, axis=1)
    pos_ls = _logsig(pos)

    dots = jax.lax.dot_general(
        c, s, (((1,), (1,)), ((), ())), preferred_element_type=jnp.float32
    )
    neg_ls = _logsig(-dots)
    col = jax.lax.broadcasted_iota(jnp.int32, neg_ls.shape, 1)
    neg_ls = jnp.where(col < _NS, neg_ls, 0.0)

    part = jnp.sum(pos_ls) + jnp.sum(neg_ls)

    @pl.when(i == 0)
    def _():
        out_ref[0, 0] = 0.0

    out_ref[0, 0] += -part

    @pl.when(i == (_B // _BLK) - 1)
    def _():
        out_ref[0, 0] = out_ref[0, 0] * (1.0 / _B)


def kernel(center, context, neg_idxs, emb_table):
    idx = neg_idxs.astype(jnp.int32)
    idx = jnp.concatenate([idx, idx[:1]]).reshape(1, _NPAD)
    samples = jnp.take(emb_table, idx[0], axis=0)

    nb = _B // _BLK
    out = pl.pallas_call(
        _loss_body,
        grid=(nb,),
        in_specs=[
            pl.BlockSpec((_BLK, _D), lambda i: (i, 0)),
            pl.BlockSpec((_BLK, _D), lambda i: (i, 0)),
            pl.BlockSpec((_NPAD, _D), lambda i: (0, 0)),
        ],
        out_specs=pl.BlockSpec(
            (1, 1), lambda i: (0, 0), memory_space=pltpu.SMEM
        ),
        out_shape=jax.ShapeDtypeStruct((1, 1), jnp.float32),
    )(center, context, samples)
    return out[0, 0]


# speedup vs baseline: 2.5673x; 1.7894x over previous
import jax
import jax.numpy as jnp
from jax.experimental import pallas as pl
from jax.experimental.pallas import tpu as pltpu
from jax.experimental.pallas import tpu_sc as plsc

_B = 16384
_D = 64
_NS = 15
_NPAD = 16
_BLK = 2048
_LANE = 128
_W = _NPAD * _LANE
_NCORES = 2


def _sc_gather_tiles(embT, tile_idx):
    mesh = plsc.ScalarSubcoreMesh(axis_name="core", num_cores=_NCORES)
    per_core = _NPAD // _NCORES

    @pl.kernel(
        out_type=jax.ShapeDtypeStruct((_D, _W), embT.dtype),
        mesh=mesh,
        scratch_types=[
            pltpu.SMEM((1, _NPAD), jnp.int32),
            pltpu.SemaphoreType.DMA,
        ],
    )
    def gather_kernel(emb_hbm, idx_hbm, out_hbm, idx_smem, sem):
        core = jax.lax.axis_index("core")
        pltpu.async_copy(idx_hbm, idx_smem, sem).wait()
        copies = []
        for n in range(per_core):
            slot = core * per_core + n
            base = pl.multiple_of(idx_smem[0, slot] * _LANE, _LANE)
            cp = pltpu.make_async_copy(
                emb_hbm.at[:, pl.ds(base, _LANE)],
                out_hbm.at[:, pl.ds(slot * _LANE, _LANE)],
                sem,
            )
            cp.start()
            copies.append(cp)
        for cp in copies:
            cp.wait()

    return gather_kernel(embT, tile_idx)


def _logsig(x):
    return jnp.minimum(x, 0.0) - jnp.log1p(jnp.exp(-jnp.abs(x)))


def _loss_body(centerT_ref, contextT_ref, wide_ref, target_ref, out_ref,
               s2_ref):
    i = pl.program_id(0)

    @pl.when(i == 0)
    def _():
        w = wide_ref[...]
        w = jnp.where(jnp.abs(w) < jnp.float32(1e30), w, 0.0)
        tgt = target_ref[...]
        j = jax.lax.broadcasted_iota(jnp.int32, (_NPAD, _W), 1)
        p = jnp.where(j == tgt, 1.0, 0.0)
        s2_ref[...] = jax.lax.dot_general(
            p, w, (((1,), (1,)), ((), ())),
            preferred_element_type=jnp.float32,
        )

    c = centerT_ref[...]
    t = contextT_ref[...]
    s = s2_ref[...]

    pos = jnp.sum(c * t, axis=0, keepdims=True)
    pos_ls = _logsig(pos)

    dots = jax.lax.dot_general(
        s, c, (((1,), (0,)), ((), ())), preferred_element_type=jnp.float32
    )
    neg_ls = _logsig(-dots)
    row = jax.lax.broadcasted_iota(jnp.int32, neg_ls.shape, 0)
    neg_ls = jnp.where(row < _NS, neg_ls, 0.0)

    part = jnp.sum(pos_ls) + jnp.sum(neg_ls)

    @pl.when(i == 0)
    def _():
        out_ref[0, 0] = 0.0

    out_ref[0, 0] += -part

    @pl.when(i == (_B // _BLK) - 1)
    def _():
        out_ref[0, 0] = out_ref[0, 0] * (1.0 / _B)


def kernel(center, context, neg_idxs, emb_table):
    idx = neg_idxs.astype(jnp.int32)
    idx = jnp.concatenate([idx, idx[:1]])
    tile_idx = (idx // _LANE).reshape(1, _NPAD)
    target = (idx % _LANE + _LANE * jnp.arange(_NPAD, dtype=jnp.int32))
    target = target.reshape(_NPAD, 1)

    cT = center.T
    tT = context.T
    eT = emb_table.T
    wide = _sc_gather_tiles(eT, tile_idx)

    nb = _B // _BLK
    out = pl.pallas_call(
        _loss_body,
        grid=(nb,),
        in_specs=[
            pl.BlockSpec((_D, _BLK), lambda i: (0, i)),
            pl.BlockSpec((_D, _BLK), lambda i: (0, i)),
            pl.BlockSpec((_D, _W), lambda i: (0, 0)),
            pl.BlockSpec((_NPAD, 1), lambda i: (0, 0)),
        ],
        out_specs=pl.BlockSpec(
            (1, 1), lambda i: (0, 0), memory_space=pltpu.SMEM
        ),
        out_shape=jax.ShapeDtypeStruct((1, 1), jnp.float32),
        scratch_shapes=[pltpu.VMEM((_NPAD, _D), jnp.float32)],
    )(cT, tT, wide, target)
    return out[0, 0]
